# baseline (speedup 1.0000x reference)
import functools

import jax
import jax.numpy as jnp
from jax import lax
from jax.experimental import pallas as pl
from jax.experimental.pallas import tpu as pltpu
from jax.experimental.pallas import tpu_sc as plsc

VOCAB = 100000
DIM = 300
DS = 128
B = 1024
L = 200

NC = 2
NS = 16
NW = NC * NS

N = B * L
N_PER_W = N // NW
CHUNK = 128
N_CHUNKS = N_PER_W // CHUNK

ASM_BR = 1024


def _make_sc_gather():
  mesh = plsc.VectorSubcoreMesh(core_axis_name="c", subcore_axis_name="s")

  @functools.partial(
      pl.kernel,
      mesh=mesh,
      compiler_params=pltpu.CompilerParams(use_tc_tiling_on_sc=False),
      out_type=tuple(jax.ShapeDtypeStruct((N, DS), jnp.float32)
                     for _ in range(3)),
      scratch_types=[
          pltpu.VMEM((N_CHUNKS, CHUNK), jnp.int32),
          pltpu.VMEM((CHUNK, DS), jnp.float32),
          pltpu.VMEM((CHUNK, DS), jnp.float32),
          pltpu.VMEM((CHUNK, DS), jnp.float32),
          pltpu.SemaphoreType.DMA,
          pltpu.SemaphoreType.DMA,
          pltpu.SemaphoreType.DMA,
      ],
  )
  def sc_gather(ta_hbm, tb_hbm, tc_hbm, idx_hbm, oa_hbm, ob_hbm, oc_hbm,
                idx_v, va, vb, vc, sa, sb, sc):
    wid = lax.axis_index("s") * NC + lax.axis_index("c")
    base = wid * N_PER_W
    pltpu.sync_copy(idx_hbm.at[wid], idx_v)

    def body(c, carry):
      rows = pl.ds(base + c * CHUNK, CHUNK)
      ix = idx_v.at[c]
      a = pltpu.async_copy(ta_hbm.at[ix], va, sa)
      b = pltpu.async_copy(tb_hbm.at[ix], vb, sb)
      d = pltpu.async_copy(tc_hbm.at[ix], vc, sc)
      a.wait()
      pltpu.sync_copy(va, oa_hbm.at[rows])
      b.wait()
      pltpu.sync_copy(vb, ob_hbm.at[rows])
      d.wait()
      pltpu.sync_copy(vc, oc_hbm.at[rows])
      return carry

    lax.fori_loop(0, N_CHUNKS, body, 0)

  return sc_gather


_sc_gather = _make_sc_gather()


def _asm_body(a_ref, b_ref, c_ref, out_ref):
  out_ref[:, 0:DS] = a_ref[...]
  out_ref[:, DS:2 * DS] = b_ref[...]
  out_ref[:, 2 * DS:DIM] = c_ref[:, :DIM - 2 * DS]


_assemble = pl.pallas_call(
    _asm_body,
    grid=(N // ASM_BR,),
    in_specs=[pl.BlockSpec((ASM_BR, DS), lambda i: (i, 0))] * 3,
    out_specs=pl.BlockSpec((ASM_BR, DIM), lambda i: (i, 0)),
    out_shape=jax.ShapeDtypeStruct((N, DIM), jnp.float32),
)


def kernel(embedding_table, word_sequences):
  ta = embedding_table[:, :DS]
  tb = embedding_table[:, DS:2 * DS]
  tc = jnp.pad(embedding_table[:, 2 * DS:], ((0, 0), (0, 3 * DS - DIM)))
  idx = word_sequences.reshape(NW, N_CHUNKS, CHUNK)
  oa, ob, oc = _sc_gather(ta, tb, tc, idx)
  out = _assemble(oa, ob, oc)
  return out.reshape(B, L, DIM)


# speedup vs baseline: 1.0721x; 1.0721x over previous
import functools

import jax
import jax.numpy as jnp
from jax import lax
from jax.experimental import pallas as pl
from jax.experimental.pallas import tpu as pltpu
from jax.experimental.pallas import tpu_sc as plsc

VOCAB = 100000
DIM = 300
DM = 256
DT = 128
B = 1024
L = 200

NC = 2
NS = 16
NW = NC * NS

N = B * L
N_PER_W = N // NW
CHUNK = 128
N_CHUNKS = N_PER_W // CHUNK


def _make_sc_gather():
  mesh = plsc.VectorSubcoreMesh(core_axis_name="c", subcore_axis_name="s")

  @functools.partial(
      pl.kernel,
      mesh=mesh,
      compiler_params=pltpu.CompilerParams(use_tc_tiling_on_sc=True),
      out_type=(jax.ShapeDtypeStruct((N, DM), jnp.float32),
                jax.ShapeDtypeStruct((N, DT), jnp.float32)),
      scratch_types=[
          pltpu.VMEM((N_CHUNKS, CHUNK), jnp.int32),
          pltpu.VMEM((CHUNK, DM), jnp.float32),
          pltpu.VMEM((CHUNK, DT), jnp.float32),
          pltpu.SemaphoreType.DMA,
          pltpu.SemaphoreType.DMA,
      ],
  )
  def sc_gather(table_hbm, tail_hbm, idx_hbm, out_hbm, outt_hbm,
                idx_v, main_v, tail_v, sem_a, sem_b):
    wid = lax.axis_index("s") * NC + lax.axis_index("c")
    base = wid * N_PER_W
    pltpu.sync_copy(idx_hbm.at[wid], idx_v)

    def body(c, carry):
      rows = pl.ds(base + c * CHUNK, CHUNK)
      a = pltpu.async_copy(
          table_hbm.at[idx_v.at[c], pl.ds(0, DM)], main_v, sem_a)
      b = pltpu.async_copy(tail_hbm.at[idx_v.at[c]], tail_v, sem_b)
      a.wait()
      pltpu.sync_copy(main_v, out_hbm.at[rows])
      b.wait()
      pltpu.sync_copy(tail_v, outt_hbm.at[rows])
      return carry

    lax.fori_loop(0, N_CHUNKS, body, 0)

  return sc_gather


_sc_gather = _make_sc_gather()


def kernel(embedding_table, word_sequences):
  tail_p = jnp.pad(embedding_table[:, DM:], ((0, 0), (0, DT - (DIM - DM))))
  idx = word_sequences.reshape(NW, N_CHUNKS, CHUNK)
  out, outt = _sc_gather(embedding_table, tail_p, idx)
  out = jnp.concatenate([out, outt[:, :DIM - DM]], axis=1)
  return out.reshape(B, L, DIM)


# speedup vs baseline: 1.0743x; 1.0021x over previous
import functools

import jax
import jax.numpy as jnp
from jax import lax
from jax.experimental import pallas as pl
from jax.experimental.pallas import tpu as pltpu
from jax.experimental.pallas import tpu_sc as plsc

VOCAB = 100000
DIM = 300
DM = 256
DT = 128
B = 1024
L = 200

NC = 2
NS = 16
NW = NC * NS

N = B * L
N_PER_W = N // NW
CHUNK = 128
N_CHUNKS = N_PER_W // CHUNK


def _make_sc_gather():
  mesh = plsc.VectorSubcoreMesh(core_axis_name="c", subcore_axis_name="s")

  @functools.partial(
      pl.kernel,
      mesh=mesh,
      compiler_params=pltpu.CompilerParams(use_tc_tiling_on_sc=True),
      out_type=(jax.ShapeDtypeStruct((N, DM), jnp.float32),
                jax.ShapeDtypeStruct((N, DT), jnp.float32)),
      scratch_types=[
          pltpu.VMEM((N_CHUNKS, CHUNK), jnp.int32),
          pltpu.VMEM((CHUNK, DM), jnp.float32),
          pltpu.VMEM((CHUNK, DT), jnp.float32),
          pltpu.SemaphoreType.DMA,
          pltpu.SemaphoreType.DMA,
      ],
  )
  def sc_gather(table_hbm, tail_hbm, idx_hbm, out_hbm, outt_hbm,
                idx_v, main_v, tail_v, sem_a, sem_b):
    wid = lax.axis_index("s") * NC + lax.axis_index("c")
    base = wid * N_PER_W
    pltpu.sync_copy(idx_hbm.at[wid], idx_v)

    def body(c, carry):
      rows = pl.ds(base + c * CHUNK, CHUNK)
      a = pltpu.async_copy(
          table_hbm.at[idx_v.at[c], pl.ds(0, DM)], main_v, sem_a)
      b = pltpu.async_copy(tail_hbm.at[idx_v.at[c]], tail_v, sem_b)
      a.wait()
      pltpu.sync_copy(main_v, out_hbm.at[rows])
      b.wait()
      pltpu.sync_copy(tail_v, outt_hbm.at[rows])
      return carry

    lax.fori_loop(0, N_CHUNKS, body, 0)

  return sc_gather


_sc_gather = _make_sc_gather()


def kernel(embedding_table, word_sequences):
  tail_p = jnp.pad(embedding_table[:, DM:], ((0, 0), (0, DT - (DIM - DM))))
  idx = word_sequences.reshape(NW, N_CHUNKS, CHUNK)
  out, outt = _sc_gather(embedding_table, tail_p, idx)
  out = jnp.concatenate([out, outt[:, :DIM - DM]], axis=1)
  return out


# speedup vs baseline: 1.2891x; 1.2000x over previous
import functools

import jax
import jax.numpy as jnp
from jax import lax
from jax.experimental import pallas as pl
from jax.experimental.pallas import tpu as pltpu
from jax.experimental.pallas import tpu_sc as plsc

VOCAB = 100000
DIM = 300
DM = 256
DT = 128
B = 1024
L = 200

NC = 2
NS = 16
NW = NC * NS

N = B * L
N_PER_W = N // NW
CHUNK = 128
N_CHUNKS = N_PER_W // CHUNK


def _make_sc_gather():
  mesh = plsc.VectorSubcoreMesh(core_axis_name="c", subcore_axis_name="s")

  @functools.partial(
      pl.kernel,
      mesh=mesh,
      compiler_params=pltpu.CompilerParams(use_tc_tiling_on_sc=True),
      out_type=(jax.ShapeDtypeStruct((N, DIM), jnp.float32),
                jax.ShapeDtypeStruct((N, DT), jnp.float32)),
      scratch_types=[
          pltpu.VMEM((N_CHUNKS, CHUNK), jnp.int32),
          pltpu.VMEM((CHUNK, DM), jnp.float32),
          pltpu.VMEM((CHUNK, DM), jnp.float32),
          pltpu.VMEM((CHUNK, DT), jnp.float32),
          pltpu.VMEM((CHUNK, DT), jnp.float32),
          pltpu.SemaphoreType.DMA,
          pltpu.SemaphoreType.DMA,
          pltpu.SemaphoreType.DMA,
          pltpu.SemaphoreType.DMA,
          pltpu.SemaphoreType.DMA,
          pltpu.SemaphoreType.DMA,
          pltpu.SemaphoreType.DMA,
          pltpu.SemaphoreType.DMA,
      ],
  )
  def sc_gather(table_hbm, tail_hbm, idx_hbm, out_hbm, outt_hbm,
                idx_v, va0, va1, vb0, vb1,
                sa0, sa1, sb0, sb1, oa0, oa1, ob0, ob1):
    va = (va0, va1)
    vb = (vb0, vb1)
    sa = (sa0, sa1)
    sb = (sb0, sb1)
    oa = (oa0, oa1)
    ob = (ob0, ob1)
    wid = lax.axis_index("s") * NC + lax.axis_index("c")
    base = wid * N_PER_W
    pltpu.sync_copy(idx_hbm.at[wid], idx_v)

    def start_gather(c, k):
      pltpu.async_copy(table_hbm.at[idx_v.at[c], pl.ds(0, DM)], va[k], sa[k])
      pltpu.async_copy(tail_hbm.at[idx_v.at[c]], vb[k], sb[k])

    def wait_gather(c, k):
      pltpu.make_async_copy(
          table_hbm.at[idx_v.at[c], pl.ds(0, DM)], va[k], sa[k]).wait()
      pltpu.make_async_copy(tail_hbm.at[idx_v.at[c]], vb[k], sb[k]).wait()

    def start_out(c, k):
      rows = pl.ds(base + c * CHUNK, CHUNK)
      pltpu.async_copy(va[k], out_hbm.at[rows, pl.ds(0, DM)], oa[k])
      pltpu.async_copy(vb[k], outt_hbm.at[rows], ob[k])

    def wait_out(c, k):
      rows = pl.ds(base + c * CHUNK, CHUNK)
      pltpu.make_async_copy(va[k], out_hbm.at[rows, pl.ds(0, DM)], oa[k]).wait()
      pltpu.make_async_copy(vb[k], outt_hbm.at[rows], ob[k]).wait()

    start_gather(0, 0)
    start_gather(1, 1)

    def body(r, carry):
      for k in (0, 1):
        c = 2 * r + k
        wait_gather(c, k)
        start_out(c, k)
      for k in (0, 1):
        c = 2 * r + k

        @pl.when(c + 2 < N_CHUNKS)
        def _():
          wait_out(c, k)
          start_gather(c + 2, k)

      return carry

    lax.fori_loop(0, N_CHUNKS // 2, body, 0)
    wait_out(N_CHUNKS - 2, 0)
    wait_out(N_CHUNKS - 1, 1)

  return sc_gather


_sc_gather = _make_sc_gather()


def kernel(embedding_table, word_sequences):
  tail_p = jnp.pad(embedding_table[:, DM:], ((0, 0), (0, DT - (DIM - DM))))
  idx = word_sequences.reshape(NW, N_CHUNKS, CHUNK)
  out, outt = _sc_gather(embedding_table, tail_p, idx)
  out = lax.dynamic_update_slice(out, outt[:, :DIM - DM], (0, DM))
  return out.reshape(B, L, DIM)


# speedup vs baseline: 1.3015x; 1.0097x over previous
import functools

import jax
import jax.numpy as jnp
from jax import lax
from jax.experimental import pallas as pl
from jax.experimental.pallas import tpu as pltpu
from jax.experimental.pallas import tpu_sc as plsc

VOCAB = 100000
DIM = 300
DM = 256
DT = 128
B = 1024
L = 200

NC = 2
NS = 16
NW = NC * NS

N = B * L
N_PER_W = N // NW
CHUNK = 64
N_CHUNKS = N_PER_W // CHUNK
NBUF = 4


def _make_sc_gather():
  mesh = plsc.VectorSubcoreMesh(core_axis_name="c", subcore_axis_name="s")

  @functools.partial(
      pl.kernel,
      mesh=mesh,
      compiler_params=pltpu.CompilerParams(use_tc_tiling_on_sc=True),
      out_type=(jax.ShapeDtypeStruct((N, DIM), jnp.float32),
                jax.ShapeDtypeStruct((N, DT), jnp.float32)),
      scratch_types=(
          [pltpu.VMEM((N_CHUNKS, CHUNK), jnp.int32)]
          + [pltpu.VMEM((CHUNK, DM), jnp.float32)] * NBUF
          + [pltpu.VMEM((CHUNK, DT), jnp.float32)] * NBUF
          + [pltpu.SemaphoreType.DMA] * (4 * NBUF)
      ),
  )
  def sc_gather(table_hbm, tail_hbm, idx_hbm, out_hbm, outt_hbm,
                idx_v, *bufs_and_sems):
    va = bufs_and_sems[0:NBUF]
    vb = bufs_and_sems[NBUF:2 * NBUF]
    sa = bufs_and_sems[2 * NBUF:3 * NBUF]
    sb = bufs_and_sems[3 * NBUF:4 * NBUF]
    oa = bufs_and_sems[4 * NBUF:5 * NBUF]
    ob = bufs_and_sems[5 * NBUF:6 * NBUF]
    wid = lax.axis_index("s") * NC + lax.axis_index("c")
    base = wid * N_PER_W
    pltpu.sync_copy(idx_hbm.at[wid], idx_v)

    def start_gather(c, k):
      pltpu.async_copy(table_hbm.at[idx_v.at[c], pl.ds(0, DM)], va[k], sa[k])
      pltpu.async_copy(tail_hbm.at[idx_v.at[c]], vb[k], sb[k])

    def wait_gather(c, k):
      pltpu.make_async_copy(
          table_hbm.at[idx_v.at[c], pl.ds(0, DM)], va[k], sa[k]).wait()
      pltpu.make_async_copy(tail_hbm.at[idx_v.at[c]], vb[k], sb[k]).wait()

    def start_out(c, k):
      rows = pl.ds(base + c * CHUNK, CHUNK)
      pltpu.async_copy(va[k], out_hbm.at[rows, pl.ds(0, DM)], oa[k])
      pltpu.async_copy(vb[k], outt_hbm.at[rows], ob[k])

    def wait_out(c, k):
      rows = pl.ds(base + c * CHUNK, CHUNK)
      pltpu.make_async_copy(va[k], out_hbm.at[rows, pl.ds(0, DM)], oa[k]).wait()
      pltpu.make_async_copy(vb[k], outt_hbm.at[rows], ob[k]).wait()

    for k in range(NBUF):
      start_gather(k, k)

    def body(r, carry):
      for k in range(NBUF):
        c = NBUF * r + k
        wait_gather(c, k)
        start_out(c, k)
      for k in range(NBUF):
        c = NBUF * r + k

        @pl.when(c + NBUF < N_CHUNKS)
        def _():
          wait_out(c, k)
          start_gather(c + NBUF, k)

      return carry

    lax.fori_loop(0, N_CHUNKS // NBUF, body, 0)
    for k in range(NBUF):
      wait_out(N_CHUNKS - NBUF + k, k)

  return sc_gather


_sc_gather = _make_sc_gather()


def kernel(embedding_table, word_sequences):
  tail_p = jnp.pad(embedding_table[:, DM:], ((0, 0), (0, DT - (DIM - DM))))
  idx = word_sequences.reshape(NW, N_CHUNKS, CHUNK)
  out, outt = _sc_gather(embedding_table, tail_p, idx)
  out = lax.dynamic_update_slice(out, outt[:, :DIM - DM], (0, DM))
  return out.reshape(B, L, DIM)
